# Initial kernel scaffold; baseline (speedup 1.0000x reference)
import jax
import jax.numpy as jnp
from jax.experimental import pallas as pl


def kernel(transcriptomic_data, x, edge_index, batch, fc1_W, fc1_b, adj_mask, W_root, W_nb, prop_b, fc2_W, fc2_b):
    raise NotImplementedError("write your pallas kernel here")


# speedup vs baseline: 120.8724x; 120.8724x over previous
import functools

import jax
import jax.numpy as jnp
from jax import lax
from jax.experimental import pallas as pl
from jax.experimental.pallas import tpu as pltpu
from jax.experimental.pallas import tpu_sc as plsc

_B = 8
_NN = 10000
_NA = 4000
_NG = 2048
_NC = 10
_N = _B * _NN
_E = _N * 32
_KK = 5000

_NA_PAD = 4096
_NN_PAD = 10240
_FC1_BLK = 512
_FC1_GRID = _NN_PAD // _FC1_BLK
_FC1_MM_STEPS = _NA_PAD // _FC1_BLK

_NW = 32
_ROWS = _E // 128
_ROWS_PER_W = _ROWS // _NW
_CHUNKS = 25
_ROWS_PER_CHUNK = _ROWS_PER_W // _CHUNKS
_NODES_PER_TILE = _N // 16


def _fc1_body(td_ref, w_ref, m_ref, b_ref, xg_ref, o_ref):
    i = pl.program_id(0)

    @pl.when(i < _FC1_MM_STEPS)
    def _():
        wm = w_ref[...] * m_ref[...]
        mm = lax.dot_general(
            td_ref[...], wm, (((1,), (1,)), ((), ())),
            preferred_element_type=jnp.float32) + b_ref[...]
        col = i * _FC1_BLK + lax.broadcasted_iota(jnp.int32, (_B, _FC1_BLK), 1)
        o_ref[...] = jnp.where(col < _NA, mm, xg_ref[...])

    @pl.when(i >= _FC1_MM_STEPS)
    def _():
        o_ref[...] = xg_ref[...]


def _fc1(td, fc1_W_pad, adj_mask_pad, b2d_pad, xg_pad):
    return pl.pallas_call(
        _fc1_body,
        grid=(_FC1_GRID,),
        in_specs=[
            pl.BlockSpec((_B, _NG), lambda i: (0, 0)),
            pl.BlockSpec((_FC1_BLK, _NG), lambda i: (jnp.minimum(i, _FC1_MM_STEPS - 1), 0)),
            pl.BlockSpec((_FC1_BLK, _NG), lambda i: (jnp.minimum(i, _FC1_MM_STEPS - 1), 0)),
            pl.BlockSpec((1, _FC1_BLK), lambda i: (0, jnp.minimum(i, _FC1_MM_STEPS - 1))),
            pl.BlockSpec((_B, _FC1_BLK), lambda i: (0, i)),
        ],
        out_specs=pl.BlockSpec((_B, _FC1_BLK), lambda i: (0, i)),
        out_shape=jax.ShapeDtypeStruct((_B, _NN_PAD), jnp.float32),
    )(td, fc1_W_pad, adj_mask_pad, b2d_pad, xg_pad)


def _sc_agg(xflat, src3, dst3, zeros, ones):
    mesh = plsc.VectorSubcoreMesh(core_axis_name="c", subcore_axis_name="s")

    @functools.partial(
        pl.kernel,
        out_type=[
            jax.ShapeDtypeStruct((2, _N), jnp.float32),
            jax.ShapeDtypeStruct((2, _N), jnp.float32),
        ],
        mesh=mesh,
        scratch_types=[
            pltpu.VMEM((_N,), jnp.float32),
            pltpu.VMEM((_ROWS_PER_CHUNK, 128), jnp.int32),
            pltpu.VMEM((_ROWS_PER_CHUNK, 128), jnp.int32),
            pltpu.VMEM((_ROWS_PER_CHUNK, 128), jnp.float32),
            pltpu.VMEM((128,), jnp.float32),
            pltpu.VMEM_SHARED((_N,), jnp.float32),
            pltpu.VMEM_SHARED((_N,), jnp.float32),
            pltpu.SemaphoreType.DMA,
        ],
        compiler_params=pltpu.CompilerParams(
            use_tc_tiling_on_sc=False, needs_layout_passes=False),
    )
    def k(x_hbm, src_hbm, dst_hbm, z_hbm, o_hbm, agg_hbm, deg_hbm,
          x_tab, src_v, dst_v, val_v, ones_v, agg_sh, deg_sh, sem):
        c = lax.axis_index("c")
        s = lax.axis_index("s")
        w = s * 2 + c

        zslice = pl.ds(s * _NODES_PER_TILE, _NODES_PER_TILE)
        pltpu.sync_copy(z_hbm.at[zslice], agg_sh.at[zslice])
        pltpu.sync_copy(z_hbm.at[zslice], deg_sh.at[zslice])
        pltpu.sync_copy(x_hbm, x_tab)
        pltpu.sync_copy(o_hbm, ones_v)
        plsc.subcore_barrier()

        gbase = w * _ROWS_PER_W

        def chunk(g, carry):
            off = gbase + g * _ROWS_PER_CHUNK
            pltpu.sync_copy(src_hbm.at[pl.ds(off, _ROWS_PER_CHUNK)], src_v)
            pltpu.sync_copy(dst_hbm.at[pl.ds(off, _ROWS_PER_CHUNK)], dst_v)
            for a in range(_ROWS_PER_CHUNK):
                for j in range(8):
                    idx = src_v[a, pl.ds(j * 16, 16)]
                    val_v[a, pl.ds(j * 16, 16)] = plsc.load_gather(x_tab, [idx])
            descs = []
            for a in range(_ROWS_PER_CHUNK):
                descs.append(pltpu.async_copy(
                    val_v.at[a], agg_sh.at[dst_v.at[a]], sem, add=True))
                descs.append(pltpu.async_copy(
                    ones_v, deg_sh.at[dst_v.at[a]], sem, add=True))
            for dsc in descs:
                dsc.wait()
            return carry

        lax.fori_loop(0, _CHUNKS, chunk, 0)
        plsc.subcore_barrier()

        pltpu.sync_copy(agg_sh.at[zslice], agg_hbm.at[c, zslice])
        pltpu.sync_copy(deg_sh.at[zslice], deg_hbm.at[c, zslice])

    return k(xflat, src3, dst3, zeros, ones)


def _finish_body(aggs_ref, degs_ref, xg_ref, consts_ref, w2_ref, b2_ref, o_ref):
    agg = aggs_ref[0] + aggs_ref[1]
    deg = degs_ref[0] + degs_ref[1]
    mean = agg / jnp.maximum(deg, 1.0)
    wr = consts_ref[0, 0]
    wn = consts_ref[0, 1]
    pb = consts_ref[0, 2]
    h = xg_ref[...] * wr + mean * wn + pb
    a = jnp.abs(h)
    ai = lax.bitcast_convert_type(a, jnp.int32)

    t = jnp.zeros((_B, 1), jnp.int32)
    for bit in range(30, -1, -1):
        cand = t | (1 << bit)
        cnt = jnp.sum((ai >= cand).astype(jnp.int32), axis=1, keepdims=True)
        t = jnp.where(cnt >= _KK, cand, t)

    gt = ai > t
    eq = ai == t
    cnt_gt = jnp.sum(gt.astype(jnp.int32), axis=1, keepdims=True)
    need = _KK - cnt_gt
    col = lax.broadcasted_iota(jnp.int32, (_B, _NN), 1)
    m = jnp.zeros((_B, 1), jnp.int32)
    for bit in range(13, -1, -1):
        cand = m | (1 << bit)
        cm = jnp.sum((eq & (col < cand)).astype(jnp.int32), axis=1, keepdims=True)
        m = jnp.where(cm <= need, cand, m)

    mask = gt | (eq & (col < m))
    dense = jnp.where(mask, h, 0.0)
    o_ref[...] = lax.dot_general(
        dense, w2_ref[...], (((1,), (1,)), ((), ())),
        preferred_element_type=jnp.float32) + b2_ref[...]


def _finish(aggs, degs, xg, consts, fc2_W, b2d):
    return pl.pallas_call(
        _finish_body,
        in_specs=[
            pl.BlockSpec(memory_space=pltpu.MemorySpace.VMEM),
            pl.BlockSpec(memory_space=pltpu.MemorySpace.VMEM),
            pl.BlockSpec(memory_space=pltpu.MemorySpace.VMEM),
            pl.BlockSpec(memory_space=pltpu.MemorySpace.SMEM),
            pl.BlockSpec(memory_space=pltpu.MemorySpace.VMEM),
            pl.BlockSpec(memory_space=pltpu.MemorySpace.VMEM),
        ],
        out_specs=pl.BlockSpec(memory_space=pltpu.MemorySpace.VMEM),
        out_shape=jax.ShapeDtypeStruct((_B, _NC), jnp.float32),
    )(aggs, degs, xg, consts, fc2_W, b2d)


def kernel(transcriptomic_data, x, edge_index, batch, fc1_W, fc1_b, adj_mask,
           W_root, W_nb, prop_b, fc2_W, fc2_b):
    xg = x.reshape(_B, _NN)
    src3 = edge_index[0].reshape(_ROWS, 128)
    dst3 = edge_index[1].reshape(_ROWS, 128)

    w_pad = jnp.pad(fc1_W, ((0, _NA_PAD - _NA), (0, 0)))
    m_pad = jnp.pad(adj_mask, ((0, _NA_PAD - _NA), (0, 0)))
    b_pad = jnp.pad(fc1_b, (0, _NA_PAD - _NA)).reshape(1, _NA_PAD)
    xg_pad = jnp.pad(xg, ((0, 0), (0, _NN_PAD - _NN)))
    x_new = _fc1(transcriptomic_data, w_pad, m_pad, b_pad, xg_pad)[:, :_NN]
    xflat = x_new.reshape(_N)

    zeros = jnp.zeros((_N,), jnp.float32)
    ones = jnp.ones((128,), jnp.float32)
    agg2, deg2 = _sc_agg(xflat, src3, dst3, zeros, ones)

    consts = jnp.stack([W_root.reshape(()), W_nb.reshape(()),
                        prop_b.reshape(())]).reshape(1, 3)
    out = _finish(agg2.reshape(2, _B, _NN), deg2.reshape(2, _B, _NN),
                  x_new, consts, fc2_W, fc2_b.reshape(1, _NC))
    return out


# speedup vs baseline: 147.3563x; 1.2191x over previous
import functools

import jax
import jax.numpy as jnp
from jax import lax
from jax.experimental import pallas as pl
from jax.experimental.pallas import tpu as pltpu
from jax.experimental.pallas import tpu_sc as plsc

_B = 8
_NN = 10000
_NA = 4000
_NG = 2048
_NC = 10
_N = _B * _NN
_E = _N * 32
_KK = 5000

_FC1_BLK = 400
_FC1_GRID = _NN // _FC1_BLK
_FC1_MM_STEPS = _NA // _FC1_BLK

_NW = 32
_ROWS = _E // 128
_ROWS_PER_W = _ROWS // _NW
_CHUNKS = 25
_ROWS_PER_CHUNK = _ROWS_PER_W // _CHUNKS
_NODES_PER_TILE = _N // 16


def _fc1_body(td_ref, w_ref, m_ref, b_ref, xgt_ref, o_ref):
    i = pl.program_id(0)

    @pl.when(i < _FC1_MM_STEPS)
    def _():
        wm = w_ref[...] * m_ref[...]
        o_ref[...] = lax.dot_general(
            wm, td_ref[...], (((1,), (1,)), ((), ())),
            preferred_element_type=jnp.float32) + b_ref[...]

    @pl.when(i >= _FC1_MM_STEPS)
    def _():
        o_ref[...] = xgt_ref[...]


def _fc1(td, fc1_W, adj_mask, bcol, xgt):
    return pl.pallas_call(
        _fc1_body,
        grid=(_FC1_GRID,),
        in_specs=[
            pl.BlockSpec((_B, _NG), lambda i: (0, 0)),
            pl.BlockSpec((_FC1_BLK, _NG), lambda i: (jnp.minimum(i, _FC1_MM_STEPS - 1), 0)),
            pl.BlockSpec((_FC1_BLK, _NG), lambda i: (jnp.minimum(i, _FC1_MM_STEPS - 1), 0)),
            pl.BlockSpec((_FC1_BLK, 1), lambda i: (jnp.minimum(i, _FC1_MM_STEPS - 1), 0)),
            pl.BlockSpec((_FC1_BLK, _B), lambda i: (i, 0)),
        ],
        out_specs=pl.BlockSpec((_FC1_BLK, _B), lambda i: (i, 0)),
        out_shape=jax.ShapeDtypeStruct((_NN, _B), jnp.float32),
    )(td, fc1_W, adj_mask, bcol, xgt)


def _sc_agg(xflat, ei2, zeros, ones):
    mesh = plsc.VectorSubcoreMesh(core_axis_name="c", subcore_axis_name="s")

    @functools.partial(
        pl.kernel,
        out_type=[
            jax.ShapeDtypeStruct((2, _N), jnp.float32),
            jax.ShapeDtypeStruct((2, _N), jnp.float32),
        ],
        mesh=mesh,
        scratch_types=[
            pltpu.VMEM((_N,), jnp.float32),
            pltpu.VMEM((_ROWS_PER_CHUNK, 128), jnp.int32),
            pltpu.VMEM((_ROWS_PER_CHUNK, 128), jnp.int32),
            pltpu.VMEM((_ROWS_PER_CHUNK, 128), jnp.float32),
            pltpu.VMEM((128,), jnp.float32),
            pltpu.VMEM_SHARED((_N,), jnp.float32),
            pltpu.VMEM_SHARED((_N,), jnp.float32),
            pltpu.SemaphoreType.DMA,
        ],
        compiler_params=pltpu.CompilerParams(
            use_tc_tiling_on_sc=False, needs_layout_passes=False),
    )
    def k(x_hbm, ei_hbm, z_hbm, o_hbm, agg_hbm, deg_hbm,
          x_tab, src_v, dst_v, val_v, ones_v, agg_sh, deg_sh, sem):
        c = lax.axis_index("c")
        s = lax.axis_index("s")
        w = s * 2 + c

        zslice = pl.ds(s * _NODES_PER_TILE, _NODES_PER_TILE)
        pltpu.sync_copy(z_hbm.at[zslice], agg_sh.at[zslice])
        pltpu.sync_copy(z_hbm.at[zslice], deg_sh.at[zslice])
        pltpu.sync_copy(x_hbm, x_tab)
        pltpu.sync_copy(o_hbm, ones_v)
        plsc.subcore_barrier()

        rbase = w * _ROWS_PER_W

        def chunk(g, carry):
            off = rbase + g * _ROWS_PER_CHUNK
            pltpu.sync_copy(ei_hbm.at[pl.ds(off, _ROWS_PER_CHUNK)], src_v)
            pltpu.sync_copy(ei_hbm.at[pl.ds(_ROWS + off, _ROWS_PER_CHUNK)], dst_v)
            for a in range(_ROWS_PER_CHUNK):
                for j in range(8):
                    idx = src_v[a, pl.ds(j * 16, 16)]
                    val_v[a, pl.ds(j * 16, 16)] = plsc.load_gather(x_tab, [idx])
            descs = []
            for a in range(_ROWS_PER_CHUNK):
                descs.append(pltpu.async_copy(
                    val_v.at[a], agg_sh.at[dst_v.at[a]], sem, add=True))
                descs.append(pltpu.async_copy(
                    ones_v, deg_sh.at[dst_v.at[a]], sem, add=True))
            for dsc in descs:
                dsc.wait()
            return carry

        lax.fori_loop(0, _CHUNKS, chunk, 0)
        plsc.subcore_barrier()

        pltpu.sync_copy(agg_sh.at[zslice], agg_hbm.at[c, zslice])
        pltpu.sync_copy(deg_sh.at[zslice], deg_hbm.at[c, zslice])

    return k(xflat, ei2, zeros, ones)


def _finish_body(aggs_ref, degs_ref, xg_ref, consts_ref, w2_ref, b2_ref, o_ref):
    agg = aggs_ref[0] + aggs_ref[1]
    deg = degs_ref[0] + degs_ref[1]
    mean = agg / jnp.maximum(deg, 1.0)
    wr = consts_ref[0, 0]
    wn = consts_ref[0, 1]
    pb = consts_ref[0, 2]
    h = xg_ref[...] * wr + mean * wn + pb
    a = jnp.abs(h)
    ai = lax.bitcast_convert_type(a, jnp.int32)

    t = jnp.zeros((_B, 1), jnp.int32)
    for bit in range(30, -1, -1):
        cand = t | (1 << bit)
        cnt = jnp.sum((ai >= cand).astype(jnp.int32), axis=1, keepdims=True)
        t = jnp.where(cnt >= _KK, cand, t)

    gt = ai > t
    eq = ai == t
    cnt_gt = jnp.sum(gt.astype(jnp.int32), axis=1, keepdims=True)
    need = _KK - cnt_gt
    col = lax.broadcasted_iota(jnp.int32, (_B, _NN), 1)
    m = jnp.zeros((_B, 1), jnp.int32)
    for bit in range(13, -1, -1):
        cand = m | (1 << bit)
        cm = jnp.sum((eq & (col < cand)).astype(jnp.int32), axis=1, keepdims=True)
        m = jnp.where(cm <= need, cand, m)

    mask = gt | (eq & (col < m))
    dense = jnp.where(mask, h, 0.0)
    o_ref[...] = lax.dot_general(
        dense, w2_ref[...], (((1,), (1,)), ((), ())),
        preferred_element_type=jnp.float32) + b2_ref[...]


def _finish(aggs, degs, xg, consts, fc2_W, b2d):
    return pl.pallas_call(
        _finish_body,
        in_specs=[
            pl.BlockSpec(memory_space=pltpu.MemorySpace.VMEM),
            pl.BlockSpec(memory_space=pltpu.MemorySpace.VMEM),
            pl.BlockSpec(memory_space=pltpu.MemorySpace.VMEM),
            pl.BlockSpec(memory_space=pltpu.MemorySpace.SMEM),
            pl.BlockSpec(memory_space=pltpu.MemorySpace.VMEM),
            pl.BlockSpec(memory_space=pltpu.MemorySpace.VMEM),
        ],
        out_specs=pl.BlockSpec(memory_space=pltpu.MemorySpace.VMEM),
        out_shape=jax.ShapeDtypeStruct((_B, _NC), jnp.float32),
    )(aggs, degs, xg, consts, fc2_W, b2d)


def kernel(transcriptomic_data, x, edge_index, batch, fc1_W, fc1_b, adj_mask,
           W_root, W_nb, prop_b, fc2_W, fc2_b):
    xgt = x.reshape(_B, _NN).T
    ei2 = edge_index.reshape(2 * _ROWS, 128)

    x_new_t = _fc1(transcriptomic_data, fc1_W, adj_mask,
                   fc1_b.reshape(_NA, 1), xgt)
    x_new = x_new_t.T
    xflat = x_new.reshape(_N)

    zeros = jnp.zeros((_N,), jnp.float32)
    ones = jnp.ones((128,), jnp.float32)
    agg2, deg2 = _sc_agg(xflat, ei2, zeros, ones)
    aggs = agg2.reshape(2, _B, _NN)
    degs = deg2.reshape(2, _B, _NN)

    consts = jnp.stack([W_root.reshape(()), W_nb.reshape(()),
                        prop_b.reshape(())]).reshape(1, 3)
    out = _finish(aggs, degs, x_new, consts, fc2_W, fc2_b.reshape(1, _NC))
    return out
